# baseline (speedup 1.0000x reference)
import jax
import jax.numpy as jnp
from jax import lax
from jax.experimental import pallas as pl
from jax.experimental.pallas import tpu as pltpu

N = 325
NU = 64
IN_DIM = 2
IS = IN_DIM + NU
ISP = 72
ALPHA = 0.05
NUM_MAT = 5
BB = 8


def _prep_w(W, out_dim):
    a = ALPHA
    coef = jnp.array([1.0, 1.0, (1 - a) * a, (1 - a) ** 2 * a, (1 - a) ** 3],
                     dtype=jnp.float32)
    Wp = W.reshape(IS, NUM_MAT, out_dim) * coef[None, :, None]
    Wp = jnp.concatenate(
        [Wp[IN_DIM:], Wp[:IN_DIM],
         jnp.zeros((ISP - IS, NUM_MAT, out_dim), jnp.float32)], axis=0)
    return Wp.transpose(1, 0, 2).reshape(NUM_MAT * ISP, out_dim).T


def _mmt(a, b):
    return lax.dot_general(a, b, (((1,), (1,)), ((), ())),
                           preferred_element_type=jnp.float32)


def _body(inp_ref, hx_ref, adp_ref, supt_ref, wrut_ref, wct_ref, out_ref):
    supt = supt_ref[...]
    for j in range(BB):
        adp = adp_ref[j]
        hxt = hx_ref[j]
        inpt = jnp.pad(inp_ref[j], ((0, ISP - NU - IN_DIM), (0, 0)))
        xt = jnp.concatenate([hxt, inpt], axis=0)

        def hops(x):
            s = jnp.dot(x, supt, preferred_element_type=jnp.float32)
            y1 = _mmt(x, adp)
            y2 = _mmt(y1, adp)
            y3 = _mmt(y2, adp)
            return jnp.concatenate([x, s, y1, y2, y3], axis=0)

        ru = jax.nn.sigmoid(
            jnp.dot(wrut_ref[...], hops(xt),
                    preferred_element_type=jnp.float32))
        r = ru[:NU]
        u = ru[NU:]
        x2 = jnp.concatenate([r * hxt, xt[NU:ISP]], axis=0)
        c = jnp.tanh(
            jnp.dot(wct_ref[...], hops(x2),
                    preferred_element_type=jnp.float32))
        out_ref[j] = u * hxt + (1.0 - u) * c


def kernel(inputs, hx, time_axis, adp, support, W_ru, W_c):
    B = inputs.shape[0]
    wrut = _prep_w(W_ru, 2 * NU)
    wct = _prep_w(W_c, NU)
    out = pl.pallas_call(
        _body,
        grid=(B // BB,),
        in_specs=[
            pl.BlockSpec((BB, IN_DIM, N), lambda b: (b, 0, 0)),
            pl.BlockSpec((BB, NU, N), lambda b: (b, 0, 0)),
            pl.BlockSpec((BB, N, N), lambda b: (b, 0, 0)),
            pl.BlockSpec((N, N), lambda b: (0, 0)),
            pl.BlockSpec((2 * NU, NUM_MAT * ISP), lambda b: (0, 0)),
            pl.BlockSpec((NU, NUM_MAT * ISP), lambda b: (0, 0)),
        ],
        out_specs=pl.BlockSpec((BB, NU, N), lambda b: (b, 0, 0)),
        out_shape=jax.ShapeDtypeStruct((B, NU, N), jnp.float32),
        compiler_params=pltpu.CompilerParams(
            dimension_semantics=("parallel",)),
    )(inputs.reshape(B, N, IN_DIM).transpose(0, 2, 1),
      hx.reshape(B, N, NU).transpose(0, 2, 1), adp, support.T, wrut, wct)
    return out.transpose(0, 2, 1).reshape(B, N * NU)


# speedup vs baseline: 1.1127x; 1.1127x over previous
import jax
import jax.numpy as jnp
from jax import lax
from jax.experimental import pallas as pl
from jax.experimental.pallas import tpu as pltpu

N = 325
NU = 64
IN_DIM = 2
IS = IN_DIM + NU
ISP = 72
ALPHA = 0.05
NUM_MAT = 5
BB = 8


def _prep_w(W, out_dim):
    a = ALPHA
    coef = jnp.array([1.0, 1.0, (1 - a) * a, (1 - a) ** 2 * a, (1 - a) ** 3],
                     dtype=jnp.float32)
    Wp = W.reshape(IS, NUM_MAT, out_dim) * coef[None, :, None]
    Wp = jnp.concatenate(
        [Wp[IN_DIM:], Wp[:IN_DIM],
         jnp.zeros((ISP - IS, NUM_MAT, out_dim), jnp.float32)], axis=0)
    return Wp.transpose(1, 0, 2).reshape(NUM_MAT * ISP, out_dim).T


def _mmt(a, b):
    return lax.dot_general(a, b, (((1,), (1,)), ((), ())),
                           preferred_element_type=jnp.float32)


def _body(inp_ref, hx_ref, adp_ref, supt_ref, wrut_ref, wct_ref, out_ref):
    supt = supt_ref[...]
    for j in range(BB):
        adp = adp_ref[j]
        hxt = hx_ref[j].T
        inpt = jnp.pad(inp_ref[j].T, ((0, ISP - IS), (0, 0)))
        xt = jnp.concatenate([hxt, inpt], axis=0)

        def hops(x):
            s = jnp.dot(x, supt, preferred_element_type=jnp.float32)
            y1 = _mmt(x, adp)
            y2 = _mmt(y1, adp)
            y3 = _mmt(y2, adp)
            return jnp.concatenate([x, s, y1, y2, y3], axis=0)

        ru = jax.nn.sigmoid(
            jnp.dot(wrut_ref[...], hops(xt),
                    preferred_element_type=jnp.float32))
        r = ru[:NU]
        u = ru[NU:]
        x2 = jnp.concatenate([r * hxt, xt[NU:ISP]], axis=0)
        c = jnp.tanh(
            jnp.dot(wct_ref[...], hops(x2),
                    preferred_element_type=jnp.float32))
        out_ref[j] = (u * hxt + (1.0 - u) * c).T


def kernel(inputs, hx, time_axis, adp, support, W_ru, W_c):
    B = inputs.shape[0]
    wrut = _prep_w(W_ru, 2 * NU)
    wct = _prep_w(W_c, NU)
    out = pl.pallas_call(
        _body,
        grid=(B // BB,),
        in_specs=[
            pl.BlockSpec((BB, N, IN_DIM), lambda b: (b, 0, 0)),
            pl.BlockSpec((BB, N, NU), lambda b: (b, 0, 0)),
            pl.BlockSpec((BB, N, N), lambda b: (b, 0, 0)),
            pl.BlockSpec((N, N), lambda b: (0, 0)),
            pl.BlockSpec((2 * NU, NUM_MAT * ISP), lambda b: (0, 0)),
            pl.BlockSpec((NU, NUM_MAT * ISP), lambda b: (0, 0)),
        ],
        out_specs=pl.BlockSpec((BB, N, NU), lambda b: (b, 0, 0)),
        out_shape=jax.ShapeDtypeStruct((B, N, NU), jnp.float32),
        compiler_params=pltpu.CompilerParams(
            dimension_semantics=("parallel",)),
    )(inputs.reshape(B, N, IN_DIM), hx.reshape(B, N, NU), adp,
      support.T, wrut, wct)
    return out.reshape(B, N * NU)
